# baseline (speedup 1.0000x reference)
import jax
import jax.numpy as jnp
from jax import lax
from jax.experimental import pallas as pl
from jax.experimental.pallas import tpu as pltpu
from jax.experimental.pallas import tpu_sc as plsc

N = 10000
E = 320000
D = 128
H = 128
L = 3
G = 64

NC = 2
NS = 16
NW = NC * NS
EPW = E // NW
BE = 128
BLOCKS = 80
EPAD = BLOCKS * BE - EPW
NA = N + 8

WB = 624
ZR = 16


def _unpack_dst(dstp_v, win, j):
    r = j // 2
    base = (j % 2) * 64
    for k in range(4):
        w = dstp_v[r, pl.ds(base + 16 * k, 16)]
        win[0, pl.ds(16 * k, 16)] = w & 0xFFFF
        win[0, pl.ds(64 + 16 * k, 16)] = w >> 16


def _seg_sum_body(h_hbm, src_hbm, dstp_hbm, out_hbm,
                  src_v, dstp_v, win, rows_a, rows_b, aggr_sh, sem,
                  gsem_a, gsem_b):
    c = lax.axis_index("c")
    s = lax.axis_index("s")
    wid = c * NS + s

    pltpu.async_copy(src_hbm.at[wid], src_v, sem)
    pltpu.async_copy(dstp_hbm.at[wid], dstp_v, sem)

    @pl.loop(0, 24)
    def _(i):
        @pl.loop(0, D // 16)
        def _(j):
            rows_a[i, pl.ds(j * 16, 16)] = jnp.zeros((16,), jnp.float32)

    zrow = rows_a.at[pl.ds(0, ZR)]

    @pl.loop(0, WB // ZR)
    def _(k):
        pltpu.async_copy(zrow, aggr_sh.at[pl.ds(s * WB + k * ZR, ZR)],
                         gsem_a)

    @pl.when(s == NS - 1)
    def _():
        pltpu.async_copy(rows_a.at[pl.ds(0, NA - NS * WB)],
                         aggr_sh.at[pl.ds(NS * WB, NA - NS * WB)], gsem_a)

    @pl.loop(0, WB // ZR)
    def _(k):
        pltpu.make_async_copy(zrow, aggr_sh.at[pl.ds(0, ZR)], gsem_a).wait()

    @pl.when(s == NS - 1)
    def _():
        pltpu.make_async_copy(rows_a.at[pl.ds(0, NA - NS * WB)],
                              aggr_sh.at[pl.ds(0, NA - NS * WB)],
                              gsem_a).wait()

    pltpu.make_async_copy(src_hbm.at[wid], src_v, sem).wait()
    pltpu.make_async_copy(dstp_hbm.at[wid], dstp_v, sem).wait()

    plsc.subcore_barrier()

    @pl.loop(0, BLOCKS)
    def _(j):
        pltpu.sync_copy(h_hbm.at[src_v.at[j]], rows_a)
        _unpack_dst(dstp_v, win, j)
        pltpu.sync_copy(rows_a, aggr_sh.at[win.at[0]], add=True)

    plsc.subcore_barrier()

    @pl.when(s < NS - 1)
    def _():
        pltpu.sync_copy(aggr_sh.at[pl.ds(s * WB, WB)],
                        out_hbm.at[c, pl.ds(s * WB, WB)])

    @pl.when(s == NS - 1)
    def _():
        pltpu.sync_copy(aggr_sh.at[pl.ds((NS - 1) * WB, N - (NS - 1) * WB)],
                        out_hbm.at[c, pl.ds((NS - 1) * WB, N - (NS - 1) * WB)])


@jax.jit
def _sc_segment_sum(h, src, dstp):
    mesh = plsc.VectorSubcoreMesh(core_axis_name="c", subcore_axis_name="s")
    k = pl.kernel(
        _seg_sum_body,
        out_type=jax.ShapeDtypeStruct((NC, N, D), jnp.float32),
        mesh=mesh,
        scratch_types=[
            pltpu.VMEM((BLOCKS, BE), jnp.int32),
            pltpu.VMEM((BLOCKS // 2, BE), jnp.int32),
            pltpu.VMEM((8, BE), jnp.int32),
            pltpu.VMEM((BE, D), jnp.float32),
            pltpu.VMEM((BE, D), jnp.float32),
            pltpu.VMEM_SHARED((NA, D), jnp.float32),
            pltpu.SemaphoreType.DMA,
            pltpu.SemaphoreType.DMA,
            pltpu.SemaphoreType.DMA,
        ],
    )
    return k(h, src, dstp)


def _mlp_body(h_ref, a_ref, eps_ref, w1_ref, b1_ref, g1_ref, be1_ref,
              w2_ref, b2_ref, go_ref, bo_ref, o_ref):
    e = eps_ref[0, 0]
    z = (1.0 + e) * h_ref[...] + a_ref[0] + a_ref[1]
    t = jnp.dot(z, w1_ref[...], preferred_element_type=jnp.float32)
    t = t + b1_ref[...]
    mu = jnp.mean(t, axis=0, keepdims=True)
    var = jnp.mean((t - mu) ** 2, axis=0, keepdims=True)
    t = g1_ref[...] * (t - mu) * lax.rsqrt(var + 1e-5) + be1_ref[...]
    t = jnp.maximum(t, 0.0)
    u = jnp.dot(t, w2_ref[...], preferred_element_type=jnp.float32)
    u = u + b2_ref[...]
    mu2 = jnp.mean(u, axis=0, keepdims=True)
    var2 = jnp.mean((u - mu2) ** 2, axis=0, keepdims=True)
    u = go_ref[...] * (u - mu2) * lax.rsqrt(var2 + 1e-5) + bo_ref[...]
    o_ref[...] = jnp.maximum(u, 0.0)


@jax.jit
def _tc_mlp(h, aggr, eps_i, w1, b1, g1, be1, w2, b2, go, bo):
    return pl.pallas_call(
        _mlp_body,
        out_shape=jax.ShapeDtypeStruct((N, H), jnp.float32),
    )(h, aggr, eps_i, w1, b1, g1, be1, w2, b2, go, bo)


def _pool_body(rep_ref, batch_ref, o_ref):
    gid = lax.broadcasted_iota(jnp.int32, (G, N), 0)
    onehot = jnp.where(gid == batch_ref[...], 1.0, 0.0)
    o_ref[...] = jnp.dot(onehot, rep_ref[...],
                         preferred_element_type=jnp.float32)


@jax.jit
def _tc_pool(rep, batch2d):
    return pl.pallas_call(
        _pool_body,
        out_shape=jax.ShapeDtypeStruct((G, rep.shape[1]), jnp.float32),
    )(rep, batch2d)


def kernel(x, edge_index, batch, eps, W1, b1, g1, be1, W2, b2, go, bo):
    src = jnp.concatenate(
        [edge_index[0].reshape(NW, EPW),
         jnp.zeros((NW, EPAD), jnp.int32)], axis=1).reshape(NW, BLOCKS, BE)
    dstw = jnp.concatenate(
        [edge_index[1].reshape(NW, EPW),
         jnp.full((NW, EPAD), N, jnp.int32)], axis=1).reshape(NW, BLOCKS, BE)
    dstp = (dstw[:, :, :64] | (dstw[:, :, 64:] << 16)).reshape(
        NW, BLOCKS // 2, BE)
    batch2d = batch.reshape(1, N)
    h = x
    reps = []
    for i in range(L):
        aggr = _sc_segment_sum(h, src, dstp)
        h = _tc_mlp(h, aggr, eps[i].reshape(1, 1), W1[i],
                    b1[i].reshape(1, H), g1[i].reshape(1, H),
                    be1[i].reshape(1, H), W2[i], b2[i].reshape(1, H),
                    go[i].reshape(1, H), bo[i].reshape(1, H))
        reps.append(h)
    node_rep = jnp.concatenate(reps, axis=1)
    graph_rep = _tc_pool(node_rep, batch2d)
    return (graph_rep, node_rep)


# speedup vs baseline: 1.0996x; 1.0996x over previous
import jax
import jax.numpy as jnp
from jax import lax
from jax.experimental import pallas as pl
from jax.experimental.pallas import tpu as pltpu
from jax.experimental.pallas import tpu_sc as plsc

N = 10000
E = 320000
D = 128
H = 128
L = 3
G = 64

NC = 2
NS = 16
NW = NC * NS
EPW = E // NW
BE = 128
BLOCKS = 80
PB = BLOCKS // 2
EPAD = BLOCKS * BE - EPW
NA = N + 8 * NS

WB = 624
ZR = 16


def _seg_sum_body(h_hbm, src_hbm, dst_hbm, out_hbm,
                  src_v, dst_v, rows_a, rows_b, aggr_sh, sem,
                  gsem_a, gsem_b):
    c = lax.axis_index("c")
    s = lax.axis_index("s")
    wid = c * NS + s

    pltpu.async_copy(src_hbm.at[wid], src_v, sem)
    pltpu.async_copy(dst_hbm.at[wid, pl.ds(0, PB)], dst_v, sem)

    @pl.loop(0, ZR)
    def _(i):
        @pl.loop(0, D // 16)
        def _(j):
            rows_a[i, pl.ds(j * 16, 16)] = jnp.zeros((16,), jnp.float32)

    zrow = rows_a.at[pl.ds(0, ZR)]

    @pl.loop(0, WB // ZR)
    def _(k):
        pltpu.async_copy(zrow, aggr_sh.at[pl.ds(s * WB + k * ZR, ZR)],
                         gsem_a)

    @pl.when(s == NS - 1)
    def _():
        pltpu.async_copy(zrow, aggr_sh.at[pl.ds(NS * WB, N - NS * WB)],
                         gsem_a)

    @pl.loop(0, WB // ZR)
    def _(k):
        pltpu.make_async_copy(zrow, aggr_sh.at[pl.ds(0, ZR)], gsem_a).wait()

    @pl.when(s == NS - 1)
    def _():
        pltpu.make_async_copy(zrow, aggr_sh.at[pl.ds(0, ZR)], gsem_a).wait()

    pltpu.make_async_copy(src_hbm.at[wid], src_v, sem).wait()
    pltpu.make_async_copy(dst_hbm.at[wid, pl.ds(0, PB)], dst_v, sem).wait()

    plsc.subcore_barrier()

    for p in (0, 1):
        if p == 1:
            pltpu.sync_copy(dst_hbm.at[wid, pl.ds(PB, PB)], dst_v)
        base = p * PB
        pltpu.async_copy(h_hbm.at[src_v.at[base]], rows_a, gsem_a)

        @pl.loop(0, PB, step=2)
        def _(j0, base=base):
            pltpu.make_async_copy(h_hbm.at[src_v.at[base + j0]], rows_a,
                                  gsem_a).wait()
            pltpu.async_copy(h_hbm.at[src_v.at[base + j0 + 1]], rows_b,
                             gsem_b)
            pltpu.sync_copy(rows_a, aggr_sh.at[dst_v.at[j0]], add=True)

            pltpu.make_async_copy(h_hbm.at[src_v.at[base + j0]], rows_b,
                                  gsem_b).wait()

            @pl.when(j0 + 2 < PB)
            def _():
                pltpu.async_copy(h_hbm.at[src_v.at[base + j0 + 2]], rows_a,
                                 gsem_a)

            pltpu.sync_copy(rows_b, aggr_sh.at[dst_v.at[j0 + 1]], add=True)

    plsc.subcore_barrier()

    @pl.when(s < NS - 1)
    def _():
        pltpu.sync_copy(aggr_sh.at[pl.ds(s * WB, WB)],
                        out_hbm.at[c, pl.ds(s * WB, WB)])

    @pl.when(s == NS - 1)
    def _():
        pltpu.sync_copy(aggr_sh.at[pl.ds((NS - 1) * WB, N - (NS - 1) * WB)],
                        out_hbm.at[c, pl.ds((NS - 1) * WB, N - (NS - 1) * WB)])


@jax.jit
def _sc_segment_sum(h, src, dst):
    mesh = plsc.VectorSubcoreMesh(core_axis_name="c", subcore_axis_name="s")
    k = pl.kernel(
        _seg_sum_body,
        out_type=jax.ShapeDtypeStruct((NC, N, D), jnp.float32),
        mesh=mesh,
        scratch_types=[
            pltpu.VMEM((BLOCKS, BE), jnp.int32),
            pltpu.VMEM((PB, BE), jnp.int32),
            pltpu.VMEM((BE, D), jnp.float32),
            pltpu.VMEM((BE, D), jnp.float32),
            pltpu.VMEM_SHARED((NA, D), jnp.float32),
            pltpu.SemaphoreType.DMA,
            pltpu.SemaphoreType.DMA,
            pltpu.SemaphoreType.DMA,
        ],
    )
    return k(h, src, dst)


def _mlp_body(h_ref, a_ref, eps_ref, w1_ref, b1_ref, g1_ref, be1_ref,
              w2_ref, b2_ref, go_ref, bo_ref, o_ref):
    e = eps_ref[0, 0]
    z = (1.0 + e) * h_ref[...] + a_ref[0] + a_ref[1]
    t = jnp.dot(z, w1_ref[...], preferred_element_type=jnp.float32)
    t = t + b1_ref[...]
    mu = jnp.mean(t, axis=0, keepdims=True)
    var = jnp.mean((t - mu) ** 2, axis=0, keepdims=True)
    t = g1_ref[...] * (t - mu) * lax.rsqrt(var + 1e-5) + be1_ref[...]
    t = jnp.maximum(t, 0.0)
    u = jnp.dot(t, w2_ref[...], preferred_element_type=jnp.float32)
    u = u + b2_ref[...]
    mu2 = jnp.mean(u, axis=0, keepdims=True)
    var2 = jnp.mean((u - mu2) ** 2, axis=0, keepdims=True)
    u = go_ref[...] * (u - mu2) * lax.rsqrt(var2 + 1e-5) + bo_ref[...]
    o_ref[...] = jnp.maximum(u, 0.0)


@jax.jit
def _tc_mlp(h, aggr, eps_i, w1, b1, g1, be1, w2, b2, go, bo):
    return pl.pallas_call(
        _mlp_body,
        out_shape=jax.ShapeDtypeStruct((N, H), jnp.float32),
    )(h, aggr, eps_i, w1, b1, g1, be1, w2, b2, go, bo)


def _pool_body(rep_ref, batch_ref, o_ref):
    gid = lax.broadcasted_iota(jnp.int32, (G, N), 0)
    onehot = jnp.where(gid == batch_ref[...], 1.0, 0.0)
    o_ref[...] = jnp.dot(onehot, rep_ref[...],
                         preferred_element_type=jnp.float32)


@jax.jit
def _tc_pool(rep, batch2d):
    return pl.pallas_call(
        _pool_body,
        out_shape=jax.ShapeDtypeStruct((G, rep.shape[1]), jnp.float32),
    )(rep, batch2d)


def kernel(x, edge_index, batch, eps, W1, b1, g1, be1, W2, b2, go, bo):
    src = jnp.concatenate(
        [edge_index[0].reshape(NW, EPW),
         jnp.zeros((NW, EPAD), jnp.int32)], axis=1).reshape(NW, BLOCKS, BE)
    w_ids = jnp.arange(NW, dtype=jnp.int32) % NS
    pad_dst = (N + 8 * w_ids[:, None]
               + (jnp.arange(EPAD, dtype=jnp.int32) % 8)[None, :])
    dst = jnp.concatenate(
        [edge_index[1].reshape(NW, EPW), pad_dst],
        axis=1).reshape(NW, BLOCKS, BE)
    batch2d = batch.reshape(1, N)
    h = x
    reps = []
    for i in range(L):
        aggr = _sc_segment_sum(h, src, dst)
        h = _tc_mlp(h, aggr, eps[i].reshape(1, 1), W1[i],
                    b1[i].reshape(1, H), g1[i].reshape(1, H),
                    be1[i].reshape(1, H), W2[i], b2[i].reshape(1, H),
                    go[i].reshape(1, H), bo[i].reshape(1, H))
        reps.append(h)
    node_rep = jnp.concatenate(reps, axis=1)
    graph_rep = _tc_pool(node_rep, batch2d)
    return (graph_rep, node_rep)


# speedup vs baseline: 3.0632x; 2.7858x over previous
import jax
import jax.numpy as jnp
from jax import lax
from jax.experimental import pallas as pl
from jax.experimental.pallas import tpu as pltpu
from jax.experimental.pallas import tpu_sc as plsc

N = 10000
E = 320000
D = 128
H = 128
L = 3
G = 64

NC = 2
NS = 16
NW = NC * NS
EPW = E // NW
BE = 125
BLOCKS = 80
PB = BLOCKS // 2
NA = N

WB = 624
ZR = 16


def _seg_sum_body(h_hbm, src_hbm, dst_hbm, out_hbm,
                  src_v, dst_v, rows_a, rows_b, aggr_sh, sem,
                  gsem_a, gsem_b):
    c = lax.axis_index("c")
    s = lax.axis_index("s")
    wid = c * NS + s

    pltpu.async_copy(src_hbm.at[wid], src_v, sem)
    pltpu.async_copy(dst_hbm.at[wid, pl.ds(0, PB)], dst_v, sem)

    @pl.loop(0, ZR)
    def _(i):
        @pl.loop(0, D // 16)
        def _(j):
            rows_a[i, pl.ds(j * 16, 16)] = jnp.zeros((16,), jnp.float32)

    zrow = rows_a.at[pl.ds(0, ZR)]

    @pl.loop(0, WB // ZR)
    def _(k):
        pltpu.async_copy(zrow, aggr_sh.at[pl.ds(s * WB + k * ZR, ZR)],
                         gsem_a)

    @pl.when(s == NS - 1)
    def _():
        pltpu.async_copy(zrow, aggr_sh.at[pl.ds(NS * WB, N - NS * WB)],
                         gsem_a)

    @pl.loop(0, WB // ZR)
    def _(k):
        pltpu.make_async_copy(zrow, aggr_sh.at[pl.ds(0, ZR)], gsem_a).wait()

    @pl.when(s == NS - 1)
    def _():
        pltpu.make_async_copy(zrow, aggr_sh.at[pl.ds(0, ZR)], gsem_a).wait()

    pltpu.make_async_copy(src_hbm.at[wid], src_v, sem).wait()
    pltpu.make_async_copy(dst_hbm.at[wid, pl.ds(0, PB)], dst_v, sem).wait()

    plsc.subcore_barrier()

    for p in (0, 1):
        if p == 1:
            pltpu.sync_copy(dst_hbm.at[wid, pl.ds(PB, PB)], dst_v)
        base = p * PB
        pltpu.async_copy(h_hbm.at[src_v.at[base]], rows_a, gsem_a)

        @pl.loop(0, PB, step=2)
        def _(j0, base=base):
            pltpu.make_async_copy(h_hbm.at[src_v.at[base + j0]], rows_a,
                                  gsem_a).wait()
            pltpu.async_copy(h_hbm.at[src_v.at[base + j0 + 1]], rows_b,
                             gsem_b)
            pltpu.sync_copy(rows_a, aggr_sh.at[dst_v.at[j0]], add=True)

            pltpu.make_async_copy(h_hbm.at[src_v.at[base + j0]], rows_b,
                                  gsem_b).wait()

            @pl.when(j0 + 2 < PB)
            def _():
                pltpu.async_copy(h_hbm.at[src_v.at[base + j0 + 2]], rows_a,
                                 gsem_a)

            pltpu.sync_copy(rows_b, aggr_sh.at[dst_v.at[j0 + 1]], add=True)

    plsc.subcore_barrier()

    @pl.when(s < NS - 1)
    def _():
        pltpu.sync_copy(aggr_sh.at[pl.ds(s * WB, WB)],
                        out_hbm.at[c, pl.ds(s * WB, WB)])

    @pl.when(s == NS - 1)
    def _():
        pltpu.sync_copy(aggr_sh.at[pl.ds((NS - 1) * WB, N - (NS - 1) * WB)],
                        out_hbm.at[c, pl.ds((NS - 1) * WB, N - (NS - 1) * WB)])


@jax.jit
def _sc_segment_sum(h, src, dst):
    mesh = plsc.VectorSubcoreMesh(core_axis_name="c", subcore_axis_name="s")
    k = pl.kernel(
        _seg_sum_body,
        out_type=jax.ShapeDtypeStruct((NC, N, D), jnp.float32),
        mesh=mesh,
        scratch_types=[
            pltpu.VMEM((BLOCKS, BE), jnp.int32),
            pltpu.VMEM((PB, BE), jnp.int32),
            pltpu.VMEM((BE, D), jnp.float32),
            pltpu.VMEM((BE, D), jnp.float32),
            pltpu.VMEM_SHARED((NA, D), jnp.float32),
            pltpu.SemaphoreType.DMA,
            pltpu.SemaphoreType.DMA,
            pltpu.SemaphoreType.DMA,
        ],
    )
    return k(h, src, dst)


def _mlp_body(h_ref, a_ref, eps_ref, w1_ref, b1_ref, g1_ref, be1_ref,
              w2_ref, b2_ref, go_ref, bo_ref, o_ref):
    e = eps_ref[0, 0]
    z = (1.0 + e) * h_ref[...] + a_ref[0] + a_ref[1]
    t = jnp.dot(z, w1_ref[...], preferred_element_type=jnp.float32)
    t = t + b1_ref[...]
    mu = jnp.mean(t, axis=0, keepdims=True)
    var = jnp.mean((t - mu) ** 2, axis=0, keepdims=True)
    t = g1_ref[...] * (t - mu) * lax.rsqrt(var + 1e-5) + be1_ref[...]
    t = jnp.maximum(t, 0.0)
    u = jnp.dot(t, w2_ref[...], preferred_element_type=jnp.float32)
    u = u + b2_ref[...]
    mu2 = jnp.mean(u, axis=0, keepdims=True)
    var2 = jnp.mean((u - mu2) ** 2, axis=0, keepdims=True)
    u = go_ref[...] * (u - mu2) * lax.rsqrt(var2 + 1e-5) + bo_ref[...]
    o_ref[...] = jnp.maximum(u, 0.0)


@jax.jit
def _tc_mlp(h, aggr, eps_i, w1, b1, g1, be1, w2, b2, go, bo):
    return pl.pallas_call(
        _mlp_body,
        out_shape=jax.ShapeDtypeStruct((N, H), jnp.float32),
    )(h, aggr, eps_i, w1, b1, g1, be1, w2, b2, go, bo)


def _pool_body(rep_ref, batch_ref, o_ref):
    gid = lax.broadcasted_iota(jnp.int32, (G, N), 0)
    onehot = jnp.where(gid == batch_ref[...], 1.0, 0.0)
    o_ref[...] = jnp.dot(onehot, rep_ref[...],
                         preferred_element_type=jnp.float32)


@jax.jit
def _tc_pool(rep, batch2d):
    return pl.pallas_call(
        _pool_body,
        out_shape=jax.ShapeDtypeStruct((G, rep.shape[1]), jnp.float32),
    )(rep, batch2d)


def kernel(x, edge_index, batch, eps, W1, b1, g1, be1, W2, b2, go, bo):
    src = edge_index[0].reshape(NW, BLOCKS, BE)
    dst = edge_index[1].reshape(NW, BLOCKS, BE)
    batch2d = batch.reshape(1, N)
    h = x
    reps = []
    for i in range(L):
        aggr = _sc_segment_sum(h, src, dst)
        h = _tc_mlp(h, aggr, eps[i].reshape(1, 1), W1[i],
                    b1[i].reshape(1, H), g1[i].reshape(1, H),
                    be1[i].reshape(1, H), W2[i], b2[i].reshape(1, H),
                    go[i].reshape(1, H), bo[i].reshape(1, H))
        reps.append(h)
    node_rep = jnp.concatenate(reps, axis=1)
    graph_rep = _tc_pool(node_rep, batch2d)
    return (graph_rep, node_rep)


# speedup vs baseline: 3.1059x; 1.0140x over previous
import jax
import jax.numpy as jnp
from jax import lax
from jax.experimental import pallas as pl
from jax.experimental.pallas import tpu as pltpu
from jax.experimental.pallas import tpu_sc as plsc

N = 10000
E = 320000
D = 128
H = 128
L = 3
G = 64

NC = 2
NS = 16
NW = NC * NS
EPW = E // NW
BE = 125
BLOCKS = 80
PB = BLOCKS // 2
NA = N

WB = 624
ZR = 16


def _seg_sum_body(h_hbm, src_hbm, dst_hbm, out_hbm,
                  src_v, dst_v, rows_a, rows_b, aggr_sh, sem,
                  gsem_a, gsem_b):
    c = lax.axis_index("c")
    s = lax.axis_index("s")
    wid = c * NS + s

    pltpu.async_copy(src_hbm.at[wid], src_v, sem)
    pltpu.async_copy(dst_hbm.at[wid, pl.ds(0, PB)], dst_v, sem)

    @pl.loop(0, ZR)
    def _(i):
        @pl.loop(0, D // 16)
        def _(j):
            rows_a[i, pl.ds(j * 16, 16)] = jnp.zeros((16,), jnp.float32)

    zrow = rows_a.at[pl.ds(0, ZR)]

    @pl.loop(0, WB // ZR)
    def _(k):
        pltpu.async_copy(zrow, aggr_sh.at[pl.ds(s * WB + k * ZR, ZR)],
                         gsem_a)

    @pl.when(s == NS - 1)
    def _():
        pltpu.async_copy(zrow, aggr_sh.at[pl.ds(NS * WB, N - NS * WB)],
                         gsem_a)

    @pl.loop(0, WB // ZR)
    def _(k):
        pltpu.make_async_copy(zrow, aggr_sh.at[pl.ds(0, ZR)], gsem_a).wait()

    @pl.when(s == NS - 1)
    def _():
        pltpu.make_async_copy(zrow, aggr_sh.at[pl.ds(0, ZR)], gsem_a).wait()

    pltpu.make_async_copy(src_hbm.at[wid], src_v, sem).wait()
    pltpu.make_async_copy(dst_hbm.at[wid, pl.ds(0, PB)], dst_v, sem).wait()

    plsc.subcore_barrier()

    for p in (0, 1):
        if p == 1:
            pltpu.sync_copy(dst_hbm.at[wid, pl.ds(PB, PB)], dst_v)
        base = p * PB
        pltpu.async_copy(h_hbm.at[src_v.at[base]], rows_a, gsem_a)

        @pl.loop(0, PB, step=2)
        def _(j0, base=base):
            pltpu.make_async_copy(h_hbm.at[src_v.at[base + j0]], rows_a,
                                  gsem_a).wait()
            pltpu.async_copy(h_hbm.at[src_v.at[base + j0 + 1]], rows_b,
                             gsem_b)
            pltpu.sync_copy(rows_a, aggr_sh.at[dst_v.at[j0]], add=True)

            pltpu.make_async_copy(h_hbm.at[src_v.at[base + j0]], rows_b,
                                  gsem_b).wait()

            @pl.when(j0 + 2 < PB)
            def _():
                pltpu.async_copy(h_hbm.at[src_v.at[base + j0 + 2]], rows_a,
                                 gsem_a)

            pltpu.sync_copy(rows_b, aggr_sh.at[dst_v.at[j0 + 1]], add=True)

    plsc.subcore_barrier()

    @pl.when(s < NS - 1)
    def _():
        pltpu.sync_copy(aggr_sh.at[pl.ds(s * WB, WB)],
                        out_hbm.at[c, pl.ds(s * WB, WB)])

    @pl.when(s == NS - 1)
    def _():
        pltpu.sync_copy(aggr_sh.at[pl.ds((NS - 1) * WB, N - (NS - 1) * WB)],
                        out_hbm.at[c, pl.ds((NS - 1) * WB, N - (NS - 1) * WB)])


@jax.jit
def _sc_segment_sum(h, src, dst):
    mesh = plsc.VectorSubcoreMesh(core_axis_name="c", subcore_axis_name="s")
    k = pl.kernel(
        _seg_sum_body,
        out_type=jax.ShapeDtypeStruct((NC, N, D), jnp.float32),
        mesh=mesh,
        scratch_types=[
            pltpu.VMEM((BLOCKS, BE), jnp.int32),
            pltpu.VMEM((PB, BE), jnp.int32),
            pltpu.VMEM((BE, D), jnp.float32),
            pltpu.VMEM((BE, D), jnp.float32),
            pltpu.VMEM_SHARED((NA, D), jnp.float32),
            pltpu.SemaphoreType.DMA,
            pltpu.SemaphoreType.DMA,
            pltpu.SemaphoreType.DMA,
        ],
    )
    return k(h, src, dst)


def _mlp_body(h_ref, a_ref, eps_ref, w1_ref, b1_ref, g1_ref, be1_ref,
              w2_ref, b2_ref, go_ref, bo_ref, batch_ref, o_ref, p_ref):
    e = eps_ref[0, 0]
    z = (1.0 + e) * h_ref[...] + a_ref[0] + a_ref[1]
    t = jnp.dot(z, w1_ref[...], preferred_element_type=jnp.float32)
    t = t + b1_ref[...]
    mu = jnp.mean(t, axis=0, keepdims=True)
    var = jnp.mean((t - mu) ** 2, axis=0, keepdims=True)
    t = g1_ref[...] * (t - mu) * lax.rsqrt(var + 1e-5) + be1_ref[...]
    t = jnp.maximum(t, 0.0)
    u = jnp.dot(t, w2_ref[...], preferred_element_type=jnp.float32)
    u = u + b2_ref[...]
    mu2 = jnp.mean(u, axis=0, keepdims=True)
    var2 = jnp.mean((u - mu2) ** 2, axis=0, keepdims=True)
    u = go_ref[...] * (u - mu2) * lax.rsqrt(var2 + 1e-5) + bo_ref[...]
    hout = jnp.maximum(u, 0.0)
    o_ref[...] = hout
    gid = lax.broadcasted_iota(jnp.int32, (G, N), 0)
    onehot = jnp.where(gid == batch_ref[...], 1.0, 0.0)
    p_ref[...] = jnp.dot(onehot, hout, preferred_element_type=jnp.float32)


@jax.jit
def _tc_mlp(h, aggr, eps_i, w1, b1, g1, be1, w2, b2, go, bo, batch2d):
    return pl.pallas_call(
        _mlp_body,
        out_shape=[jax.ShapeDtypeStruct((N, H), jnp.float32),
                   jax.ShapeDtypeStruct((G, H), jnp.float32)],
    )(h, aggr, eps_i, w1, b1, g1, be1, w2, b2, go, bo, batch2d)


def kernel(x, edge_index, batch, eps, W1, b1, g1, be1, W2, b2, go, bo):
    src = edge_index[0].reshape(NW, BLOCKS, BE)
    dst = edge_index[1].reshape(NW, BLOCKS, BE)
    batch2d = batch.reshape(1, N)
    h = x
    reps = []
    pooled = []
    for i in range(L):
        aggr = _sc_segment_sum(h, src, dst)
        h, pi = _tc_mlp(h, aggr, eps[i].reshape(1, 1), W1[i],
                        b1[i].reshape(1, H), g1[i].reshape(1, H),
                        be1[i].reshape(1, H), W2[i], b2[i].reshape(1, H),
                        go[i].reshape(1, H), bo[i].reshape(1, H), batch2d)
        reps.append(h)
        pooled.append(pi)
    node_rep = jnp.concatenate(reps, axis=1)
    graph_rep = jnp.concatenate(pooled, axis=1)
    return (graph_rep, node_rep)
